# Initial kernel scaffold; baseline (speedup 1.0000x reference)
import jax
import jax.numpy as jnp
from jax.experimental import pallas as pl


def kernel(senders, receivers, edge_indices, edge_attrs, W_Q, W_K, W_V, W_E, attention):
    raise NotImplementedError("write your pallas kernel here")


# speedup vs baseline: 1.0468x; 1.0468x over previous
import functools

import jax
import jax.numpy as jnp
from jax.experimental import pallas as pl

N = 10000
E = 320000
C = 128
H = 8
HC = C // H
BD = 16


def _qkv_body(recv_ref, send_ref, wq_ref, wk_ref, wv_ref, q_ref, k_ref, v_ref):
    recv = recv_ref[...]
    send = send_ref[...]
    q_ref[...] = jnp.dot(recv, wq_ref[...], preferred_element_type=jnp.float32)
    k_ref[...] = jnp.dot(send, wk_ref[...], preferred_element_type=jnp.float32)
    v_ref[...] = jnp.dot(send, wv_ref[...], preferred_element_type=jnp.float32)


def _qkv_proj(receivers, senders, W_Q, W_K, W_V):
    out = jax.ShapeDtypeStruct((N, C), jnp.float32)
    return pl.pallas_call(
        _qkv_body,
        out_shape=(out, out, out),
    )(receivers, senders, W_Q, W_K, W_V)


def _eemb_body(ea_ref, we_ref, out_ref):
    out_ref[...] = jnp.dot(ea_ref[...], we_ref[...],
                           preferred_element_type=jnp.float32)


def _eemb_proj(edge_attrs, W_E):
    blk = 16000
    grid = E // blk
    return pl.pallas_call(
        _eemb_body,
        grid=(grid,),
        in_specs=[
            pl.BlockSpec((blk, BD), lambda i: (i, 0)),
            pl.BlockSpec((BD, C), lambda i: (0, 0)),
        ],
        out_specs=pl.BlockSpec((blk, C), lambda i: (i, 0)),
        out_shape=jax.ShapeDtypeStruct((E, C), jnp.float32),
    )(edge_attrs, W_E)


def kernel(senders, receivers, edge_indices, edge_attrs, W_Q, W_K, W_V, W_E, attention):
    src = edge_indices[0]
    dst = edge_indices[1]
    Qn, Kn, Vn = _qkv_proj(receivers, senders, W_Q, W_K, W_V)
    Eemb = _eemb_proj(edge_attrs, W_E)

    Q = Qn[dst]
    K = Kn[src]
    V = Vn[src]
    s = (Q + K + Eemb).reshape(-1, H, HC)
    s = jnp.where(s >= 0, s, 0.01 * s)
    weights = jnp.sum(attention * s, axis=2)
    m = jax.ops.segment_max(weights, dst, num_segments=N)
    wexp = jnp.exp(weights - m[dst])
    denom = jax.ops.segment_sum(wexp, dst, num_segments=N)
    soft = wexp / denom[dst]
    Vh = V.reshape(-1, H, HC)
    msgs = (soft[..., None] * Vh).reshape(-1, C)
    return jax.ops.segment_sum(msgs, dst, num_segments=N)


# speedup vs baseline: 2.1375x; 2.0419x over previous
import functools

import jax
import jax.numpy as jnp
from jax import lax
from jax.experimental import pallas as pl
from jax.experimental.pallas import tpu as pltpu
from jax.experimental.pallas import tpu_sc as plsc

N = 10000
E = 320000
C = 128
H = 8
HC = C // H
BD = 16

NWORK = 32
EDGES_PER_W = E // NWORK
EB = 40
NCHUNK = EDGES_PER_W // EB
N_PAD = 10240
ROWS_PER_SUB = N_PAD // 16
DFLAT = N_PAD * H
DPAD = EB * H - (EB * H // 128) * 128
DENROWS = (EB * H + 127) // 128
DTOT = DFLAT + 2048
DROWS_PER_SUB = DTOT // 16


def _qkv_body(recv_ref, send_ref, wq_ref, wk_ref, wv_ref, q_ref, kv_ref):
    send = send_ref[...]
    q_ref[...] = jnp.dot(recv_ref[...], wq_ref[...],
                         preferred_element_type=jnp.float32)
    kv_ref[:, :C] = jnp.dot(send, wk_ref[...],
                            preferred_element_type=jnp.float32)
    kv_ref[:, C:] = jnp.dot(send, wv_ref[...],
                            preferred_element_type=jnp.float32)


def _qkv_proj(receivers, senders, W_Q, W_K, W_V):
    return pl.pallas_call(
        _qkv_body,
        out_shape=(jax.ShapeDtypeStruct((N, C), jnp.float32),
                   jax.ShapeDtypeStruct((N, 2 * C), jnp.float32)),
    )(receivers, senders, W_Q, W_K, W_V)


def _eemb_body(ea_ref, we_ref, out_ref):
    out_ref[...] = jnp.dot(ea_ref[...], we_ref[...],
                           preferred_element_type=jnp.float32)


def _eemb_proj(edge_attrs, W_E):
    blk = 16000
    return pl.pallas_call(
        _eemb_body,
        grid=(E // blk,),
        in_specs=[
            pl.BlockSpec((blk, BD), lambda i: (i, 0)),
            pl.BlockSpec((BD, C), lambda i: (0, 0)),
        ],
        out_specs=pl.BlockSpec((blk, C), lambda i: (i, 0)),
        out_shape=jax.ShapeDtypeStruct((E, C), jnp.float32),
    )(edge_attrs, W_E)


def _edge_body(q_hbm, kv_hbm, e_hbm, src_hbm, dst_hbm, att_hbm, zero_hbm,
               zerod_hbm, outm_hbm, outd_hbm, src_v, dst_v, q_v, kv_v, e_v,
               msg_v, att_v, dv0, dv1, dv2, di0, di1, di2, accum, accum_d,
               sem_q, sem_kv, sem_sc):
    cid = lax.axis_index("c")
    sid = lax.axis_index("s")
    wid = sid * 2 + cid
    dvals = (dv0, dv1, dv2)
    didxs = (di0, di1, di2)

    row0 = sid * ROWS_PER_SUB
    drow0 = sid * DROWS_PER_SUB
    pltpu.sync_copy(zero_hbm.at[pl.ds(row0, ROWS_PER_SUB)],
                    accum.at[pl.ds(row0, ROWS_PER_SUB)])
    pltpu.sync_copy(zerod_hbm.at[pl.ds(drow0, DROWS_PER_SUB)],
                    accum_d.at[pl.ds(drow0, DROWS_PER_SUB)])
    pltpu.sync_copy(att_hbm, att_v)
    zero16 = jnp.zeros((16,), jnp.float32)
    scrap16 = jnp.full((16,), DFLAT, jnp.int32)
    for t in range(DPAD // 16):
        dvals[DENROWS - 1][pl.ds(128 - DPAD + 16 * t, 16)] = zero16
        didxs[DENROWS - 1][pl.ds(128 - DPAD + 16 * t, 16)] = scrap16
    plsc.subcore_barrier()

    lanes = lax.iota(jnp.int32, 16)
    folds = [jnp.bitwise_xor(lanes, k) for k in (8, 4, 2, 1)]
    lane_hi = jnp.where(lanes >= 8, 1, 0)
    lane_and7 = jnp.bitwise_and(lanes, 7)
    npairs = EB // 2

    def chunk(i, _):
        base = wid * EDGES_PER_W + i * EB
        pltpu.sync_copy(src_hbm.at[pl.ds(base, EB)], src_v)
        pltpu.sync_copy(dst_hbm.at[pl.ds(base, EB)], dst_v)
        cp_q = pltpu.async_copy(q_hbm.at[dst_v], q_v, sem_q)
        cp_kv = pltpu.async_copy(kv_hbm.at[src_v], kv_v, sem_kv)
        pltpu.sync_copy(e_hbm.at[pl.ds(base, EB)], e_v)

        for j in range(DENROWS):
            def gidx(g, _, dij=didxs[j]):
                dvreg = dst_v[pl.ds((g // 8) * 16, 16)]
                cg = 2 * (g % 8) + lane_hi
                dpair = dvreg.at[cg].get(mode="promise_in_bounds")
                dij[pl.ds((g % 8) * 16, 16)] = dpair * 8 + lane_and7
                return 0

            lax.fori_loop(j * 8, min((j + 1) * 8, npairs), gidx, 0)
        cp_q.wait()
        cp_kv.wait()

        for j in range(DENROWS):
            def pair(e2, _, dvj=dvals[j]):
                pv = jnp.zeros((16,), jnp.float32)
                for half in range(2):
                    ee = e2 * 2 + half
                    for h in range(H):
                        sl = pl.ds(h * HC, HC)
                        s = q_v[ee, sl] + kv_v[ee, sl] + e_v[ee, sl]
                        s = jnp.where(s >= 0.0, s, 0.01 * s)
                        wv = att_v[h, :] * s
                        for fx in folds:
                            wv = wv + wv.at[fx].get(mode="promise_in_bounds")
                        wx = jnp.exp(jnp.minimum(wv, 60.0))
                        msg_v[ee, sl] = wx * kv_v[ee, pl.ds(C + h * HC, HC)]
                        pv = pv + jnp.where(lanes == half * 8 + h, wx, 0.0)
                dvj[pl.ds((e2 % 8) * 16, 16)] = pv
                return 0

            lax.fori_loop(j * 8, min((j + 1) * 8, npairs), pair, 0)
        cp_m = pltpu.async_copy(msg_v, accum.at[dst_v], sem_sc, add=True)
        for j in range(DENROWS):
            pltpu.sync_copy(dvals[j], accum_d.at[didxs[j]], add=True)
        cp_m.wait()
        return 0

    lax.fori_loop(0, NCHUNK, chunk, 0)
    plsc.subcore_barrier()
    pltpu.sync_copy(accum.at[pl.ds(row0, ROWS_PER_SUB)],
                    outm_hbm.at[cid, pl.ds(row0, ROWS_PER_SUB)])
    pltpu.sync_copy(accum_d.at[pl.ds(drow0, DROWS_PER_SUB)],
                    outd_hbm.at[pl.ds(cid * DTOT + drow0, DROWS_PER_SUB)])


def _edge_pass(Qn, KVn, Eemb, src, dst, att2, zeros, zerosd):
    f = functools.partial(
        pl.kernel, _edge_body,
        out_type=(jax.ShapeDtypeStruct((2, N_PAD, C), jnp.float32),
                  jax.ShapeDtypeStruct((2 * DTOT,), jnp.float32)),
        mesh=plsc.VectorSubcoreMesh(core_axis_name="c", subcore_axis_name="s"),
        scratch_types=[
            pltpu.VMEM((EB,), jnp.int32),
            pltpu.VMEM((EB,), jnp.int32),
            pltpu.VMEM((EB, C), jnp.float32),
            pltpu.VMEM((EB, 2 * C), jnp.float32),
            pltpu.VMEM((EB, C), jnp.float32),
            pltpu.VMEM((EB, C), jnp.float32),
            pltpu.VMEM((H, HC), jnp.float32),
            pltpu.VMEM((128,), jnp.float32),
            pltpu.VMEM((128,), jnp.float32),
            pltpu.VMEM((128,), jnp.float32),
            pltpu.VMEM((128,), jnp.int32),
            pltpu.VMEM((128,), jnp.int32),
            pltpu.VMEM((128,), jnp.int32),
            pltpu.VMEM_SHARED((N_PAD, C), jnp.float32),
            pltpu.VMEM_SHARED((DTOT,), jnp.float32),
            pltpu.SemaphoreType.DMA,
            pltpu.SemaphoreType.DMA,
            pltpu.SemaphoreType.DMA,
        ],
    )()
    return f(Qn, KVn, Eemb, src, dst, att2, zeros, zerosd)


def _final_body(pm_ref, pd_ref, out_ref):
    msg = pm_ref[0] + pm_ref[1]
    den = pd_ref[0] + pd_ref[1]
    out_ref[...] = jnp.where(den > 0.0, msg / den, 0.0)


def _finalize(pm, pd):
    blk = 2048
    return pl.pallas_call(
        _final_body,
        grid=(N_PAD // blk,),
        in_specs=[
            pl.BlockSpec((2, blk, C), lambda i: (0, i, 0)),
            pl.BlockSpec((2, blk, C), lambda i: (0, i, 0)),
        ],
        out_specs=pl.BlockSpec((blk, C), lambda i: (i, 0)),
        out_shape=jax.ShapeDtypeStruct((N_PAD, C), jnp.float32),
    )(pm, pd)


def kernel(senders, receivers, edge_indices, edge_attrs, W_Q, W_K, W_V, W_E, attention):
    src = edge_indices[0]
    dst = edge_indices[1]
    Qn, KVn = _qkv_proj(receivers, senders, W_Q, W_K, W_V)
    Eemb = _eemb_proj(edge_attrs, W_E)
    att2 = attention.reshape(H, HC)
    zeros = jnp.zeros((N_PAD, C), jnp.float32)
    zerosd = jnp.zeros((DTOT,), jnp.float32)
    pm, pd = _edge_pass(Qn, KVn, Eemb, src, dst, att2, zeros, zerosd)
    denx = jnp.repeat(pd.reshape(2, DTOT)[:, :DFLAT].reshape(2, N_PAD, H), HC, axis=2)
    return _finalize(pm, denx)[:N]


# speedup vs baseline: 2.2920x; 1.0723x over previous
import functools

import jax
import jax.numpy as jnp
from jax import lax
from jax.experimental import pallas as pl
from jax.experimental.pallas import tpu as pltpu
from jax.experimental.pallas import tpu_sc as plsc

N = 10000
E = 320000
C = 128
H = 8
HC = C // H
BD = 16

NWORK = 32
EDGES_PER_W = E // NWORK
EB = 40
NCHUNK = EDGES_PER_W // EB
N_PAD = 10112
ROWS_PER_SUB = N_PAD // 16
DFLAT = N_PAD * H
DPAD = EB * H - (EB * H // 128) * 128
DENROWS = (EB * H + 127) // 128
DTOT = DFLAT + 1024
DROWS_PER_SUB = DTOT // 16


def _qkv_body(recv_ref, send_ref, wq_ref, wk_ref, wv_ref, q_ref, kv_ref):
    send = send_ref[...]
    q_ref[...] = jnp.dot(recv_ref[...], wq_ref[...],
                         preferred_element_type=jnp.float32)
    kv_ref[:, :C] = jnp.dot(send, wk_ref[...],
                            preferred_element_type=jnp.float32)
    kv_ref[:, C:] = jnp.dot(send, wv_ref[...],
                            preferred_element_type=jnp.float32)


def _qkv_proj(receivers, senders, W_Q, W_K, W_V):
    return pl.pallas_call(
        _qkv_body,
        out_shape=(jax.ShapeDtypeStruct((N, C), jnp.float32),
                   jax.ShapeDtypeStruct((N, 2 * C), jnp.float32)),
    )(receivers, senders, W_Q, W_K, W_V)


def _eemb_body(ea_ref, we_ref, out_ref):
    out_ref[...] = jnp.dot(ea_ref[...], we_ref[...],
                           preferred_element_type=jnp.float32)


def _eemb_proj(edge_attrs, W_E):
    blk = 16000
    return pl.pallas_call(
        _eemb_body,
        grid=(E // blk,),
        in_specs=[
            pl.BlockSpec((blk, BD), lambda i: (i, 0)),
            pl.BlockSpec((BD, C), lambda i: (0, 0)),
        ],
        out_specs=pl.BlockSpec((blk, C), lambda i: (i, 0)),
        out_shape=jax.ShapeDtypeStruct((E, C), jnp.float32),
    )(edge_attrs, W_E)


def _edge_body(q_hbm, kv_hbm, e_hbm, src_hbm, dst_hbm, att_hbm, zero_hbm,
               zerod_hbm, outm_hbm, outd_hbm,
               src0, src1, dst0, dst1, sd0, sd1,
               q0, q1, kv0, kv1, m0, m1, att_v,
               dv00, dv01, dv02, dv10, dv11, dv12,
               di00, di01, di02, di10, di11, di12,
               accum, accum_d,
               semi0, semi1, semg0, semg1, sems0, sems1):
    cid = lax.axis_index("c")
    sid = lax.axis_index("s")
    wid = sid * 2 + cid
    srcs = (src0, src1)
    dsts = (dst0, dst1)
    sdst = (sd0, sd1)
    qs = (q0, q1)
    kvs = (kv0, kv1)
    msgs = (m0, m1)
    dvs = ((dv00, dv01, dv02), (dv10, dv11, dv12))
    dis = ((di00, di01, di02), (di10, di11, di12))
    semi = (semi0, semi1)
    semg = (semg0, semg1)
    sems = (sems0, sems1)

    row0 = sid * ROWS_PER_SUB
    drow0 = sid * DROWS_PER_SUB
    pltpu.sync_copy(zero_hbm.at[pl.ds(row0, ROWS_PER_SUB)],
                    accum.at[pl.ds(row0, ROWS_PER_SUB)])
    pltpu.sync_copy(zerod_hbm.at[pl.ds(drow0, DROWS_PER_SUB)],
                    accum_d.at[pl.ds(drow0, DROWS_PER_SUB)])
    pltpu.sync_copy(att_hbm, att_v)
    zero16 = jnp.zeros((16,), jnp.float32)
    scrap16 = jnp.full((16,), DFLAT, jnp.int32)
    for p in range(2):
        for t in range(DPAD // 16):
            dvs[p][DENROWS - 1][pl.ds(128 - DPAD + 16 * t, 16)] = zero16
            dis[p][DENROWS - 1][pl.ds(128 - DPAD + 16 * t, 16)] = scrap16
    plsc.subcore_barrier()

    lanes = lax.iota(jnp.int32, 16)
    folds = [jnp.bitwise_xor(lanes, k) for k in (8, 4, 2, 1)]
    lane_hi = jnp.where(lanes >= 8, 1, 0)
    lane_and7 = jnp.bitwise_and(lanes, 7)
    npairs = EB // 2

    def issue_idx(i, p):
        base = wid * EDGES_PER_W + i * EB
        pltpu.async_copy(src_hbm.at[pl.ds(base, EB)], srcs[p], semi[p])
        pltpu.async_copy(dst_hbm.at[pl.ds(base, EB)], dsts[p], semi[p])

    def wait_idx(p):
        pltpu.make_async_copy(src_hbm.at[pl.ds(0, EB)], srcs[p], semi[p]).wait()
        pltpu.make_async_copy(dst_hbm.at[pl.ds(0, EB)], dsts[p], semi[p]).wait()

    def issue_gathers(i, p):
        base = wid * EDGES_PER_W + i * EB
        pltpu.async_copy(q_hbm.at[dsts[p]], qs[p], semg[p])
        pltpu.async_copy(kv_hbm.at[srcs[p]], kvs[p], semg[p])
        pltpu.async_copy(e_hbm.at[pl.ds(base, EB)], msgs[p], semg[p])

    def wait_gathers(p):
        pltpu.make_async_copy(q_hbm.at[dsts[p]], qs[p], semg[p]).wait()
        pltpu.make_async_copy(kv_hbm.at[srcs[p]], kvs[p], semg[p]).wait()
        pltpu.make_async_copy(e_hbm.at[pl.ds(0, EB)], msgs[p], semg[p]).wait()

    def issue_scatters(p):
        pltpu.async_copy(msgs[p], accum.at[sdst[p]], sems[p], add=True)
        for j in range(DENROWS):
            pltpu.async_copy(dvs[p][j], accum_d.at[dis[p][j]], sems[p],
                             add=True)

    def wait_scatters(p):
        pltpu.make_async_copy(msgs[p], accum.at[sdst[p]], sems[p]).wait()
        for j in range(DENROWS):
            pltpu.make_async_copy(dvs[p][j], accum_d.at[dis[p][j]],
                                  sems[p]).wait()

    def compute(p):
        q_v, kv_v, msg_v = qs[p], kvs[p], msgs[p]
        for k in range(3):
            off = (0, 16, 24)[k]
            sdst[p][pl.ds(off, 16)] = dsts[p][pl.ds(off, 16)]
        for j in range(DENROWS):
            def gidx(g, _, dij=dis[p][j]):
                dvreg = dsts[p][pl.ds((g // 8) * 16, 16)]
                cg = 2 * (g % 8) + lane_hi
                dpair = dvreg.at[cg].get(mode="promise_in_bounds")
                dij[pl.ds((g % 8) * 16, 16)] = dpair * 8 + lane_and7
                return 0

            lax.fori_loop(j * 8, min((j + 1) * 8, npairs), gidx, 0)

        for j in range(DENROWS):
            def pair(e2, _, dvj=dvs[p][j]):
                pv = jnp.zeros((16,), jnp.float32)
                for half in range(2):
                    ee = e2 * 2 + half
                    for h in range(H):
                        sl = pl.ds(h * HC, HC)
                        s = q_v[ee, sl] + kv_v[ee, sl] + msg_v[ee, sl]
                        s = jnp.where(s >= 0.0, s, 0.01 * s)
                        wv = att_v[h, :] * s
                        for fx in folds:
                            wv = wv + wv.at[fx].get(mode="promise_in_bounds")
                        wx = jnp.exp(jnp.minimum(wv, 60.0))
                        msg_v[ee, sl] = wx * kv_v[ee, pl.ds(C + h * HC, HC)]
                        pv = pv + jnp.where(lanes == half * 8 + h, wx, 0.0)
                dvj[pl.ds((e2 % 8) * 16, 16)] = pv
                return 0

            lax.fori_loop(j * 8, min((j + 1) * 8, npairs), pair, 0)

    issue_idx(0, 0)
    wait_idx(0)
    issue_gathers(0, 0)
    issue_idx(1, 1)

    def body2(t, _):
        for b in range(2):
            i = 2 * t + b
            wait_gathers(b)
            compute(b)
            issue_scatters(b)

            @pl.when(i >= 1)
            def _():
                wait_scatters(1 - b)

            @pl.when(i < NCHUNK - 1)
            def _():
                wait_idx(1 - b)
                issue_gathers(i + 1, 1 - b)

            @pl.when(i < NCHUNK - 2)
            def _():
                issue_idx(i + 2, b)
        return 0

    lax.fori_loop(0, NCHUNK // 2, body2, 0)
    wait_scatters((NCHUNK - 1) % 2)
    plsc.subcore_barrier()
    pltpu.sync_copy(accum.at[pl.ds(row0, ROWS_PER_SUB)],
                    outm_hbm.at[cid, pl.ds(row0, ROWS_PER_SUB)])
    pltpu.sync_copy(accum_d.at[pl.ds(drow0, DROWS_PER_SUB)],
                    outd_hbm.at[pl.ds(cid * DTOT + drow0, DROWS_PER_SUB)])


def _edge_pass(Qn, KVn, Eemb, src, dst, att2, zeros, zerosd):
    f = functools.partial(
        pl.kernel, _edge_body,
        out_type=(jax.ShapeDtypeStruct((2, N_PAD, C), jnp.float32),
                  jax.ShapeDtypeStruct((2 * DTOT,), jnp.float32)),
        mesh=plsc.VectorSubcoreMesh(core_axis_name="c", subcore_axis_name="s"),
        scratch_types=(
            [pltpu.VMEM((EB,), jnp.int32)] * 6
            + [pltpu.VMEM((EB, C), jnp.float32),
               pltpu.VMEM((EB, C), jnp.float32),
               pltpu.VMEM((EB, 2 * C), jnp.float32),
               pltpu.VMEM((EB, 2 * C), jnp.float32),
               pltpu.VMEM((EB, C), jnp.float32),
               pltpu.VMEM((EB, C), jnp.float32),
               pltpu.VMEM((H, HC), jnp.float32)]
            + [pltpu.VMEM((128,), jnp.float32)] * 6
            + [pltpu.VMEM((128,), jnp.int32)] * 6
            + [pltpu.VMEM_SHARED((N_PAD, C), jnp.float32),
               pltpu.VMEM_SHARED((DTOT,), jnp.float32)]
            + [pltpu.SemaphoreType.DMA] * 6
        ),
    )()
    return f(Qn, KVn, Eemb, src, dst, att2, zeros, zerosd)


def _final_body(pm_ref, pd_ref, out_ref):
    msg = pm_ref[0] + pm_ref[1]
    den = pd_ref[0] + pd_ref[1]
    out_ref[...] = jnp.where(den > 0.0, msg / den, 0.0)


def _finalize(pm, pd):
    blk = N_PAD
    return pl.pallas_call(
        _final_body,
        grid=(N_PAD // blk,),
        in_specs=[
            pl.BlockSpec((2, blk, C), lambda i: (0, i, 0)),
            pl.BlockSpec((2, blk, C), lambda i: (0, i, 0)),
        ],
        out_specs=pl.BlockSpec((blk, C), lambda i: (i, 0)),
        out_shape=jax.ShapeDtypeStruct((N_PAD, C), jnp.float32),
    )(pm, pd)


def kernel(senders, receivers, edge_indices, edge_attrs, W_Q, W_K, W_V, W_E, attention):
    src = edge_indices[0]
    dst = edge_indices[1]
    Qn, KVn = _qkv_proj(receivers, senders, W_Q, W_K, W_V)
    Eemb = _eemb_proj(edge_attrs, W_E)
    att2 = attention.reshape(H, HC)
    zeros = jnp.zeros((N_PAD, C), jnp.float32)
    zerosd = jnp.zeros((DTOT,), jnp.float32)
    pm, pd = _edge_pass(Qn, KVn, Eemb, src, dst, att2, zeros, zerosd)
    denx = jnp.repeat(pd.reshape(2, DTOT)[:, :DFLAT].reshape(2, N_PAD, H), HC, axis=2)
    return _finalize(pm, denx)[:N]


# speedup vs baseline: 9.0782x; 3.9609x over previous
import functools

import jax
import jax.numpy as jnp
from jax import lax
from jax.experimental import pallas as pl
from jax.experimental.pallas import tpu as pltpu
from jax.experimental.pallas import tpu_sc as plsc

N = 10000
E = 320000
C = 128
H = 8
HC = C // H
BD = 16

NWORK = 32
EDGES_PER_W = E // NWORK
EB = 40
NCHUNK = EDGES_PER_W // EB
N_PAD = 10112
ROWS_PER_SUB = N_PAD // 16
DFLAT = N_PAD * H
DPAD = EB * H - (EB * H // 128) * 128
DENROWS = (EB * H + 127) // 128
DTOT = DFLAT + 1024
DROWS_PER_SUB = DTOT // 16


def _qkv_body(recv_ref, send_ref, wq_ref, wk_ref, wv_ref, q_ref, kv_ref):
    send = send_ref[...]
    q_ref[...] = jnp.dot(recv_ref[...], wq_ref[...],
                         preferred_element_type=jnp.float32)
    kv_ref[:, :C] = jnp.dot(send, wk_ref[...],
                            preferred_element_type=jnp.float32)
    kv_ref[:, C:] = jnp.dot(send, wv_ref[...],
                            preferred_element_type=jnp.float32)


def _qkv_proj(receivers, senders, W_Q, W_K, W_V):
    return pl.pallas_call(
        _qkv_body,
        out_shape=(jax.ShapeDtypeStruct((N, C), jnp.float32),
                   jax.ShapeDtypeStruct((N, 2 * C), jnp.float32)),
    )(receivers, senders, W_Q, W_K, W_V)


def _eemb_body(ea_ref, we_ref, out_ref):
    out_ref[...] = jnp.dot(ea_ref[...], we_ref[...],
                           preferred_element_type=jnp.float32)


def _eemb_proj(edge_attrs, W_E):
    blk = 16000
    return pl.pallas_call(
        _eemb_body,
        grid=(E // blk,),
        in_specs=[
            pl.BlockSpec((blk, BD), lambda i: (i, 0)),
            pl.BlockSpec((BD, C), lambda i: (0, 0)),
        ],
        out_specs=pl.BlockSpec((blk, C), lambda i: (i, 0)),
        out_shape=jax.ShapeDtypeStruct((E, C), jnp.float32),
    )(edge_attrs, W_E)


def _edge_body(q_hbm, kv_hbm, e_hbm, src_hbm, dst_hbm, att_hbm, zero_hbm,
               zerod_hbm, outm_hbm, outd_hbm,
               src0, src1, dst0, dst1, sd0, sd1,
               q0, q1, kv0, kv1, m0, m1, att_v,
               dv00, dv01, dv02, dv10, dv11, dv12,
               di00, di01, di02, di10, di11, di12,
               accum, accum_d,
               semi0, semi1, semg0, semg1, sems0, sems1):
    cid = lax.axis_index("c")
    sid = lax.axis_index("s")
    wid = sid * 2 + cid
    srcs = (src0, src1)
    dsts = (dst0, dst1)
    sdst = (sd0, sd1)
    qs = (q0, q1)
    kvs = (kv0, kv1)
    msgs = (m0, m1)
    dvs = ((dv00, dv01, dv02), (dv10, dv11, dv12))
    dis = ((di00, di01, di02), (di10, di11, di12))
    semi = (semi0, semi1)
    semg = (semg0, semg1)
    sems = (sems0, sems1)

    row0 = sid * ROWS_PER_SUB
    drow0 = sid * DROWS_PER_SUB
    pltpu.sync_copy(zero_hbm.at[pl.ds(row0, ROWS_PER_SUB)],
                    accum.at[pl.ds(row0, ROWS_PER_SUB)])
    pltpu.sync_copy(zerod_hbm.at[pl.ds(drow0, DROWS_PER_SUB)],
                    accum_d.at[pl.ds(drow0, DROWS_PER_SUB)])
    pltpu.sync_copy(att_hbm, att_v)
    zero16 = jnp.zeros((16,), jnp.float32)
    scrap16 = jnp.full((16,), DFLAT, jnp.int32)
    for p in range(2):
        for t in range(DPAD // 16):
            dvs[p][DENROWS - 1][pl.ds(128 - DPAD + 16 * t, 16)] = zero16
            dis[p][DENROWS - 1][pl.ds(128 - DPAD + 16 * t, 16)] = scrap16
    plsc.subcore_barrier()

    lanes = lax.iota(jnp.int32, 16)
    folds = [jnp.bitwise_xor(lanes, k) for k in (8, 4, 2, 1)]
    lane_hi = jnp.where(lanes >= 8, 1, 0)
    lane_and7 = jnp.bitwise_and(lanes, 7)
    npairs = EB // 2

    def issue_idx(i, p):
        base = wid * EDGES_PER_W + i * EB
        pltpu.async_copy(src_hbm.at[pl.ds(base, EB)], srcs[p], semi[p])
        pltpu.async_copy(dst_hbm.at[pl.ds(base, EB)], dsts[p], semi[p])

    def wait_idx(p):
        pltpu.make_async_copy(src_hbm.at[pl.ds(0, EB)], srcs[p], semi[p]).wait()
        pltpu.make_async_copy(dst_hbm.at[pl.ds(0, EB)], dsts[p], semi[p]).wait()

    def issue_gathers(i, p):
        base = wid * EDGES_PER_W + i * EB
        pltpu.async_copy(q_hbm.at[dsts[p]], qs[p], semg[p])
        pltpu.async_copy(kv_hbm.at[srcs[p]], kvs[p], semg[p])
        pltpu.async_copy(e_hbm.at[pl.ds(base, EB)], msgs[p], semg[p])

    def wait_gathers(p):
        pltpu.make_async_copy(q_hbm.at[dsts[p]], qs[p], semg[p]).wait()
        pltpu.make_async_copy(kv_hbm.at[srcs[p]], kvs[p], semg[p]).wait()
        pltpu.make_async_copy(e_hbm.at[pl.ds(0, EB)], msgs[p], semg[p]).wait()

    def issue_scatters(p):
        pltpu.async_copy(msgs[p], accum.at[sdst[p]], sems[p], add=True)
        for j in range(DENROWS):
            pltpu.async_copy(dvs[p][j], accum_d.at[dis[p][j]], sems[p],
                             add=True)

    def wait_scatters(p):
        pltpu.make_async_copy(msgs[p], accum.at[sdst[p]], sems[p]).wait()
        for j in range(DENROWS):
            pltpu.make_async_copy(dvs[p][j], accum_d.at[dis[p][j]],
                                  sems[p]).wait()

    def compute(p):
        q_v, kv_v, msg_v = qs[p], kvs[p], msgs[p]
        for k in range(3):
            off = (0, 16, 24)[k]
            sdst[p][pl.ds(off, 16)] = dsts[p][pl.ds(off, 16)]
        for j in range(DENROWS):
            def gidx(g, _, dij=dis[p][j]):
                dvreg = dsts[p][pl.ds((g // 8) * 16, 16)]
                cg = 2 * (g % 8) + lane_hi
                dpair = dvreg.at[cg].get(mode="promise_in_bounds")
                dij[pl.ds((g % 8) * 16, 16)] = dpair * 8 + lane_and7
                return 0

            lax.fori_loop(j * 8, min((j + 1) * 8, npairs), gidx, 0)

        for j in range(DENROWS):
            def pair(e2, _, dvj=dvs[p][j]):
                pv = jnp.zeros((16,), jnp.float32)
                for half in range(2):
                    ee = e2 * 2 + half
                    for h in range(H):
                        sl = pl.ds(h * HC, HC)
                        s = q_v[ee, sl] + kv_v[ee, sl] + msg_v[ee, sl]
                        s = jnp.where(s >= 0.0, s, 0.01 * s)
                        wv = att_v[h, :] * s
                        for fx in folds:
                            wv = wv + wv.at[fx].get(mode="promise_in_bounds")
                        wx = jnp.exp(jnp.minimum(wv, 60.0))
                        msg_v[ee, sl] = wx * kv_v[ee, pl.ds(C + h * HC, HC)]
                        pv = pv + jnp.where(lanes == half * 8 + h, wx, 0.0)
                dvj[pl.ds((e2 % 8) * 16, 16)] = pv
                return 0

            if False:
                lax.fori_loop(j * 8, min((j + 1) * 8, npairs), pair, 0)

    issue_idx(0, 0)
    wait_idx(0)
    issue_gathers(0, 0)
    issue_idx(1, 1)

    def body2(t, _):
        for b in range(2):
            i = 2 * t + b
            wait_gathers(b)
            compute(b)
            issue_scatters(b)

            @pl.when(i >= 1)
            def _():
                wait_scatters(1 - b)

            @pl.when(i < NCHUNK - 1)
            def _():
                wait_idx(1 - b)
                issue_gathers(i + 1, 1 - b)

            @pl.when(i < NCHUNK - 2)
            def _():
                issue_idx(i + 2, b)
        return 0

    lax.fori_loop(0, NCHUNK // 2, body2, 0)
    wait_scatters((NCHUNK - 1) % 2)
    plsc.subcore_barrier()
    pltpu.sync_copy(accum.at[pl.ds(row0, ROWS_PER_SUB)],
                    outm_hbm.at[cid, pl.ds(row0, ROWS_PER_SUB)])
    pltpu.sync_copy(accum_d.at[pl.ds(drow0, DROWS_PER_SUB)],
                    outd_hbm.at[pl.ds(cid * DTOT + drow0, DROWS_PER_SUB)])


def _edge_pass(Qn, KVn, Eemb, src, dst, att2, zeros, zerosd):
    f = functools.partial(
        pl.kernel, _edge_body,
        out_type=(jax.ShapeDtypeStruct((2, N_PAD, C), jnp.float32),
                  jax.ShapeDtypeStruct((2 * DTOT,), jnp.float32)),
        mesh=plsc.VectorSubcoreMesh(core_axis_name="c", subcore_axis_name="s"),
        scratch_types=(
            [pltpu.VMEM((EB,), jnp.int32)] * 6
            + [pltpu.VMEM((EB, C), jnp.float32),
               pltpu.VMEM((EB, C), jnp.float32),
               pltpu.VMEM((EB, 2 * C), jnp.float32),
               pltpu.VMEM((EB, 2 * C), jnp.float32),
               pltpu.VMEM((EB, C), jnp.float32),
               pltpu.VMEM((EB, C), jnp.float32),
               pltpu.VMEM((H, HC), jnp.float32)]
            + [pltpu.VMEM((128,), jnp.float32)] * 6
            + [pltpu.VMEM((128,), jnp.int32)] * 6
            + [pltpu.VMEM_SHARED((N_PAD, C), jnp.float32),
               pltpu.VMEM_SHARED((DTOT,), jnp.float32)]
            + [pltpu.SemaphoreType.DMA] * 6
        ),
    )()
    return f(Qn, KVn, Eemb, src, dst, att2, zeros, zerosd)


def _final_body(pm_ref, pd_ref, out_ref):
    msg = pm_ref[0] + pm_ref[1]
    den = pd_ref[0] + pd_ref[1]
    out_ref[...] = jnp.where(den > 0.0, msg / den, 0.0)


def _finalize(pm, pd):
    blk = N_PAD
    return pl.pallas_call(
        _final_body,
        grid=(N_PAD // blk,),
        in_specs=[
            pl.BlockSpec((2, blk, C), lambda i: (0, i, 0)),
            pl.BlockSpec((2, blk, C), lambda i: (0, i, 0)),
        ],
        out_specs=pl.BlockSpec((blk, C), lambda i: (i, 0)),
        out_shape=jax.ShapeDtypeStruct((N_PAD, C), jnp.float32),
    )(pm, pd)


def kernel(senders, receivers, edge_indices, edge_attrs, W_Q, W_K, W_V, W_E, attention):
    src = edge_indices[0]
    dst = edge_indices[1]
    Qn, KVn = _qkv_proj(receivers, senders, W_Q, W_K, W_V)
    Eemb = _eemb_proj(edge_attrs, W_E)
    att2 = attention.reshape(H, HC)
    zeros = jnp.zeros((N_PAD, C), jnp.float32)
    zerosd = jnp.zeros((DTOT,), jnp.float32)
    pm, pd = _edge_pass(Qn, KVn, Eemb, src, dst, att2, zeros, zerosd)
    denx = jnp.repeat(pd.reshape(2, DTOT)[:, :DFLAT].reshape(2, N_PAD, H), HC, axis=2)
    return _finalize(pm, denx)[:N]


# speedup vs baseline: 9.1622x; 1.0093x over previous
import functools

import jax
import jax.numpy as jnp
from jax import lax
from jax.experimental import pallas as pl
from jax.experimental.pallas import tpu as pltpu
from jax.experimental.pallas import tpu_sc as plsc

N = 10000
E = 320000
C = 128
H = 8
HC = C // H
BD = 16

NWORK = 32
EDGES_PER_W = E // NWORK
EB = 40
NCHUNK = EDGES_PER_W // EB
N_PAD = 10112
ROWS_PER_SUB = N_PAD // 16
DFLAT = N_PAD * H
DPAD = EB * H - (EB * H // 128) * 128
DENROWS = (EB * H + 127) // 128
DTOT = DFLAT + 1024
DROWS_PER_SUB = DTOT // 16


def _qkv_body(recv_ref, send_ref, wq_ref, wk_ref, wv_ref, q_ref, kv_ref):
    send = send_ref[...]
    q_ref[...] = jnp.dot(recv_ref[...], wq_ref[...],
                         preferred_element_type=jnp.float32)
    kv_ref[:, :C] = jnp.dot(send, wk_ref[...],
                            preferred_element_type=jnp.float32)
    kv_ref[:, C:] = jnp.dot(send, wv_ref[...],
                            preferred_element_type=jnp.float32)


def _qkv_proj(receivers, senders, W_Q, W_K, W_V):
    return pl.pallas_call(
        _qkv_body,
        out_shape=(jax.ShapeDtypeStruct((N, C), jnp.float32),
                   jax.ShapeDtypeStruct((N, 2 * C), jnp.float32)),
    )(receivers, senders, W_Q, W_K, W_V)


def _eemb_body(ea_ref, we_ref, out_ref):
    out_ref[...] = jnp.dot(ea_ref[...], we_ref[...],
                           preferred_element_type=jnp.float32)


def _eemb_proj(edge_attrs, W_E):
    blk = 16000
    return pl.pallas_call(
        _eemb_body,
        grid=(E // blk,),
        in_specs=[
            pl.BlockSpec((blk, BD), lambda i: (i, 0)),
            pl.BlockSpec((BD, C), lambda i: (0, 0)),
        ],
        out_specs=pl.BlockSpec((blk, C), lambda i: (i, 0)),
        out_shape=jax.ShapeDtypeStruct((E, C), jnp.float32),
    )(edge_attrs, W_E)


def _edge_body(q_hbm, kv_hbm, e_hbm, src_hbm, dst_hbm, att_hbm, zero_hbm,
               zerod_hbm, outm_hbm, outd_hbm,
               src0, src1, dst0, dst1, sd0, sd1,
               q0, q1, kv0, kv1, m0, m1, att_v,
               dval0, dval1,
               di00, di01, di02, di10, di11, di12,
               accum, accum_d,
               semi0, semi1, semg0, semg1, sems0, sems1):
    cid = lax.axis_index("c")
    sid = lax.axis_index("s")
    wid = sid * 2 + cid
    srcs = (src0, src1)
    dsts = (dst0, dst1)
    sdst = (sd0, sd1)
    qs = (q0, q1)
    kvs = (kv0, kv1)
    msgs = (m0, m1)
    dvals = (dval0, dval1)
    dis = ((di00, di01, di02), (di10, di11, di12))
    semi = (semi0, semi1)
    semg = (semg0, semg1)
    sems = (sems0, sems1)

    row0 = sid * ROWS_PER_SUB
    drow0 = sid * DROWS_PER_SUB
    pltpu.sync_copy(zero_hbm.at[pl.ds(row0, ROWS_PER_SUB)],
                    accum.at[pl.ds(row0, ROWS_PER_SUB)])
    pltpu.sync_copy(zerod_hbm.at[pl.ds(drow0, DROWS_PER_SUB)],
                    accum_d.at[pl.ds(drow0, DROWS_PER_SUB)])
    pltpu.sync_copy(att_hbm, att_v)
    zero16 = jnp.zeros((16,), jnp.float32)
    scrap16 = jnp.full((16,), DFLAT, jnp.int32)
    for p in range(2):
        for t in range(DPAD // 16):
            dvals[p][pl.ds(EB * H + 16 * t, 16)] = zero16
            dis[p][DENROWS - 1][pl.ds(128 - DPAD + 16 * t, 16)] = scrap16
    plsc.subcore_barrier()

    lanes = lax.iota(jnp.int32, 16)
    folds = [jnp.bitwise_xor(lanes, k) for k in (8, 4, 2, 1)]
    lane_hi = jnp.where(lanes >= 8, 1, 0)
    lane_and7 = jnp.bitwise_and(lanes, 7)
    npairs = EB // 2

    def issue_idx(i, p):
        base = wid * EDGES_PER_W + i * EB
        pltpu.async_copy(src_hbm.at[pl.ds(base, EB)], srcs[p], semi[p])
        pltpu.async_copy(dst_hbm.at[pl.ds(base, EB)], dsts[p], semi[p])

    def wait_idx(p):
        pltpu.make_async_copy(src_hbm.at[pl.ds(0, EB)], srcs[p], semi[p]).wait()
        pltpu.make_async_copy(dst_hbm.at[pl.ds(0, EB)], dsts[p], semi[p]).wait()

    def issue_gathers(i, p):
        base = wid * EDGES_PER_W + i * EB
        pltpu.async_copy(q_hbm.at[dsts[p]], qs[p], semg[p])
        pltpu.async_copy(kv_hbm.at[srcs[p]], kvs[p], semg[p])
        pltpu.async_copy(e_hbm.at[pl.ds(base, EB)], msgs[p], semg[p])

    def wait_gathers(p):
        pltpu.make_async_copy(q_hbm.at[dsts[p]], qs[p], semg[p]).wait()
        pltpu.make_async_copy(kv_hbm.at[srcs[p]], kvs[p], semg[p]).wait()
        pltpu.make_async_copy(e_hbm.at[pl.ds(0, EB)], msgs[p], semg[p]).wait()

    def issue_scatters(p):
        pltpu.async_copy(msgs[p], accum.at[sdst[p]], sems[p], add=True)
        for j in range(DENROWS):
            pltpu.async_copy(dvals[p].at[pl.ds(j * 128, 128)],
                             accum_d.at[dis[p][j]], sems[p], add=True)

    def wait_scatters(p):
        pltpu.make_async_copy(msgs[p], accum.at[sdst[p]], sems[p]).wait()
        for j in range(DENROWS):
            pltpu.make_async_copy(dvals[p].at[pl.ds(j * 128, 128)],
                                  accum_d.at[dis[p][j]], sems[p]).wait()

    def compute(p):
        q_v, kv_v, msg_v = qs[p], kvs[p], msgs[p]
        for k in range(3):
            off = (0, 16, 24)[k]
            sdst[p][pl.ds(off, 16)] = dsts[p][pl.ds(off, 16)]
        for j in range(DENROWS):
            def gidx(g, _, dij=dis[p][j]):
                dvreg = dsts[p][pl.ds((g // 8) * 16, 16)]
                cg = 2 * (g % 8) + lane_hi
                dpair = dvreg.at[cg].get(mode="promise_in_bounds")
                dij[pl.ds((g % 8) * 16, 16)] = dpair * 8 + lane_and7
                return 0

            lax.fori_loop(j * 8, min((j + 1) * 8, npairs), gidx, 0)

        @functools.partial(plsc.parallel_loop, 0, npairs, unroll=2)
        def pair(e2):
            pv = jnp.zeros((16,), jnp.float32)
            for half in range(2):
                ee = e2 * 2 + half
                for h in range(H):
                    sl = pl.ds(h * HC, HC)
                    s = q_v[ee, sl] + kv_v[ee, sl] + msg_v[ee, sl]
                    s = jnp.where(s >= 0.0, s, 0.01 * s)
                    wv = att_v[h, :] * s
                    for fx in folds:
                        wv = wv + wv.at[fx].get(mode="promise_in_bounds")
                    wx = jnp.exp(jnp.minimum(wv, 60.0))
                    msg_v[ee, sl] = wx * kv_v[ee, pl.ds(C + h * HC, HC)]
                    pv = pv + jnp.where(lanes == half * 8 + h, wx, 0.0)
            dvals[p][pl.ds(e2 * 16, 16)] = pv

    issue_idx(0, 0)
    wait_idx(0)
    issue_gathers(0, 0)
    issue_idx(1, 1)

    def body2(t, _):
        for b in range(2):
            i = 2 * t + b
            wait_gathers(b)
            compute(b)
            issue_scatters(b)

            @pl.when(i >= 1)
            def _():
                wait_scatters(1 - b)

            @pl.when(i < NCHUNK - 1)
            def _():
                wait_idx(1 - b)
                issue_gathers(i + 1, 1 - b)

            @pl.when(i < NCHUNK - 2)
            def _():
                issue_idx(i + 2, b)
        return 0

    lax.fori_loop(0, NCHUNK // 2, body2, 0)
    wait_scatters((NCHUNK - 1) % 2)
    plsc.subcore_barrier()
    pltpu.sync_copy(accum.at[pl.ds(row0, ROWS_PER_SUB)],
                    outm_hbm.at[cid, pl.ds(row0, ROWS_PER_SUB)])
    pltpu.sync_copy(accum_d.at[pl.ds(drow0, DROWS_PER_SUB)],
                    outd_hbm.at[pl.ds(cid * DTOT + drow0, DROWS_PER_SUB)])


def _edge_pass(Qn, KVn, Eemb, src, dst, att2, zeros, zerosd):
    f = functools.partial(
        pl.kernel, _edge_body,
        out_type=(jax.ShapeDtypeStruct((2, N_PAD, C), jnp.float32),
                  jax.ShapeDtypeStruct((2 * DTOT,), jnp.float32)),
        mesh=plsc.VectorSubcoreMesh(core_axis_name="c", subcore_axis_name="s"),
        scratch_types=(
            [pltpu.VMEM((EB,), jnp.int32)] * 6
            + [pltpu.VMEM((EB, C), jnp.float32),
               pltpu.VMEM((EB, C), jnp.float32),
               pltpu.VMEM((EB, 2 * C), jnp.float32),
               pltpu.VMEM((EB, 2 * C), jnp.float32),
               pltpu.VMEM((EB, C), jnp.float32),
               pltpu.VMEM((EB, C), jnp.float32),
               pltpu.VMEM((H, HC), jnp.float32)]
            + [pltpu.VMEM((DENROWS * 128,), jnp.float32)] * 2
            + [pltpu.VMEM((128,), jnp.int32)] * 6
            + [pltpu.VMEM_SHARED((N_PAD, C), jnp.float32),
               pltpu.VMEM_SHARED((DTOT,), jnp.float32)]
            + [pltpu.SemaphoreType.DMA] * 6
        ),
    )()
    return f(Qn, KVn, Eemb, src, dst, att2, zeros, zerosd)


def _final_body(pm_ref, pd_ref, out_ref):
    msg = pm_ref[0] + pm_ref[1]
    den = pd_ref[0] + pd_ref[1]
    out_ref[...] = jnp.where(den > 0.0, msg / den, 0.0)


def _finalize(pm, pd):
    blk = N_PAD
    return pl.pallas_call(
        _final_body,
        grid=(N_PAD // blk,),
        in_specs=[
            pl.BlockSpec((2, blk, C), lambda i: (0, i, 0)),
            pl.BlockSpec((2, blk, C), lambda i: (0, i, 0)),
        ],
        out_specs=pl.BlockSpec((blk, C), lambda i: (i, 0)),
        out_shape=jax.ShapeDtypeStruct((N_PAD, C), jnp.float32),
    )(pm, pd)


def kernel(senders, receivers, edge_indices, edge_attrs, W_Q, W_K, W_V, W_E, attention):
    src = edge_indices[0]
    dst = edge_indices[1]
    Qn, KVn = _qkv_proj(receivers, senders, W_Q, W_K, W_V)
    Eemb = _eemb_proj(edge_attrs, W_E)
    att2 = attention.reshape(H, HC)
    zeros = jnp.zeros((N_PAD, C), jnp.float32)
    zerosd = jnp.zeros((DTOT,), jnp.float32)
    pm, pd = _edge_pass(Qn, KVn, Eemb, src, dst, att2, zeros, zerosd)
    denx = jnp.repeat(pd.reshape(2, DTOT)[:, :DFLAT].reshape(2, N_PAD, H), HC, axis=2)
    return _finalize(pm, denx)[:N]
